# baseline (speedup 1.0000x reference)
import functools

import jax
import jax.numpy as jnp
from jax import lax
from jax.experimental import pallas as pl
from jax.experimental.pallas import tpu as pltpu
from jax.experimental.pallas import tpu_sc as plsc

N = 10000
E = 320000
D = 128
H = 64
G = 64

NC = 2
NS = 16
NW = NC * NS
CH = 125
EPW = E // NW
NJ = EPW // CH
NBUF = 4
RPT = 624
TAILB = RPT * NS
TAILN = N - TAILB
DEGW = 16
ZR = 208

NB = 5
BLK = N // NB


def _sc_body(want_deg, h_hbm, src_hbm, dst_hbm, *rest):
    if want_deg:
        (s_out, deg_out, src_v, dst_v, rows_v, ones_v, zbuf, dzbuf,
         acc_sh, deg_sh, sem0, sem1, sem2, sem3) = rest
    else:
        (s_out, src_v, dst_v, rows_v, zbuf, acc_sh,
         sem0, sem1, sem2, sem3) = rest
    sems = (sem0, sem1, sem2, sem3)
    c = lax.axis_index("c")
    s = lax.axis_index("s")
    w = s * NC + c

    pltpu.sync_copy(src_hbm.at[pl.ds(w * NJ, NJ)], src_v)
    pltpu.sync_copy(dst_hbm.at[pl.ds(w * NJ, NJ)], dst_v)

    def zrow(i, _):
        for cc in range(H // 16):
            zbuf[i, pl.ds(cc * 16, 16)] = jnp.zeros((16,), jnp.float32)
        return 0
    lax.fori_loop(0, ZR, zrow, 0)
    for k in range(RPT // ZR):
        pltpu.sync_copy(zbuf, acc_sh.at[pl.ds(s * RPT + k * ZR, ZR)])

    @pl.when(s == NS - 1)
    def _():
        pltpu.sync_copy(zbuf.at[pl.ds(0, TAILN)], acc_sh.at[pl.ds(TAILB, TAILN)])
    if want_deg:
        def zdrow(i, _):
            dzbuf[i, pl.ds(0, DEGW)] = jnp.zeros((DEGW,), jnp.float32)
            return 0
        lax.fori_loop(0, ZR, zdrow, 0)
        for k in range(RPT // ZR):
            pltpu.sync_copy(dzbuf, deg_sh.at[pl.ds(s * RPT + k * ZR, ZR)])

        @pl.when(s == NS - 1)
        def _():
            pltpu.sync_copy(dzbuf.at[pl.ds(0, TAILN)],
                            deg_sh.at[pl.ds(TAILB, TAILN)])

        def orow(i, _):
            ones_v[i, pl.ds(0, DEGW)] = jnp.ones((DEGW,), jnp.float32)
            return 0
        lax.fori_loop(0, CH, orow, 0)
    plsc.subcore_barrier()

    for b in range(NBUF):
        pltpu.async_copy(h_hbm.at[src_v.at[b]], rows_v.at[b], sems[b])

    def edge_iter(jo, _):
        for b in range(NBUF):
            j = jo * NBUF + b
            pltpu.make_async_copy(h_hbm.at[src_v.at[j]], rows_v.at[b],
                                  sems[b]).wait()
            pltpu.sync_copy(rows_v.at[b], acc_sh.at[dst_v.at[j]], add=True)
            if want_deg:
                pltpu.sync_copy(ones_v, deg_sh.at[dst_v.at[j]], add=True)

            @pl.when(j + NBUF < NJ)
            def _():
                pltpu.async_copy(h_hbm.at[src_v.at[j + NBUF]],
                                 rows_v.at[b], sems[b])
        return 0
    lax.fori_loop(0, NJ // NBUF, edge_iter, 0)
    plsc.subcore_barrier()

    pltpu.sync_copy(acc_sh.at[pl.ds(s * RPT, RPT)],
                    s_out.at[c, pl.ds(s * RPT, RPT)])

    @pl.when(s == NS - 1)
    def _():
        pltpu.sync_copy(acc_sh.at[pl.ds(TAILB, TAILN)],
                        s_out.at[c, pl.ds(TAILB, TAILN)])
    if want_deg:
        pltpu.sync_copy(deg_sh.at[pl.ds(s * RPT, RPT)],
                        deg_out.at[c, pl.ds(s * RPT, RPT)])

        @pl.when(s == NS - 1)
        def _():
            pltpu.sync_copy(deg_sh.at[pl.ds(TAILB, TAILN)],
                            deg_out.at[c, pl.ds(TAILB, TAILN)])


def _sc_segment_sum(h, src2d, dst2d, want_deg):
    mesh = plsc.VectorSubcoreMesh(core_axis_name="c", subcore_axis_name="s")
    out_type = [jax.ShapeDtypeStruct((NC, N, H), jnp.float32)]
    sems = [pltpu.SemaphoreType.DMA] * NBUF
    if want_deg:
        out_type.append(jax.ShapeDtypeStruct((NC, N, DEGW), jnp.float32))
        scratch = [
            pltpu.VMEM((NJ, CH), jnp.int32),
            pltpu.VMEM((NJ, CH), jnp.int32),
            pltpu.VMEM((NBUF, CH, H), jnp.float32),
            pltpu.VMEM((CH, DEGW), jnp.float32),
            pltpu.VMEM((ZR, H), jnp.float32),
            pltpu.VMEM((ZR, DEGW), jnp.float32),
            pltpu.VMEM_SHARED((N, H), jnp.float32),
            pltpu.VMEM_SHARED((N, DEGW), jnp.float32),
        ] + sems
    else:
        scratch = [
            pltpu.VMEM((NJ, CH), jnp.int32),
            pltpu.VMEM((NJ, CH), jnp.int32),
            pltpu.VMEM((NBUF, CH, H), jnp.float32),
            pltpu.VMEM((ZR, H), jnp.float32),
            pltpu.VMEM_SHARED((N, H), jnp.float32),
        ] + sems
    fn = pl.kernel(
        functools.partial(_sc_body, want_deg),
        out_type=tuple(out_type),
        mesh=mesh,
        scratch_types=tuple(scratch),
        compiler_params=pltpu.CompilerParams(use_tc_tiling_on_sc=False),
    )
    return fn(h, src2d, dst2d)


def _kl_rows(a, wn, l):
    nrm = jnp.sqrt(jnp.sum(a * a, axis=1, keepdims=True))
    xh = a / jnp.maximum(nrm, 1e-6)
    t = lax.dot_general(xh, wn, (((1,), (1,)), ((), ())),
                        preferred_element_type=jnp.float32)
    emb = nrm * jnp.exp(t - 1.0)
    return lax.dot_general(emb, l, (((1,), (0,)), ((), ())),
                           preferred_element_type=jnp.float32)


def _tc_kl0_body(x_ref, wn_ref, l_ref, out_ref):
    out_ref[...] = _kl_rows(x_ref[...], wn_ref[...], l_ref[...])


def _tc_kl0(x, wn, l):
    return pl.pallas_call(
        _tc_kl0_body,
        grid=(NB,),
        in_specs=[
            pl.BlockSpec((BLK, D), lambda i: (i, 0)),
            pl.BlockSpec((H, D), lambda i: (0, 0)),
            pl.BlockSpec((H, H), lambda i: (0, 0)),
        ],
        out_specs=pl.BlockSpec((BLK, H), lambda i: (i, 0)),
        out_shape=jax.ShapeDtypeStruct((N, H), jnp.float32),
    )(x, wn, l)


def _tc_fie1_body(sp_ref, degp_ref, h0_ref, mu_ref, wn_ref, l_ref, out_ref):
    s = sp_ref[0] + sp_ref[1]
    deg = degp_ref[0, :, 0:1] + degp_ref[1, :, 0:1]
    a = (s - deg * mu_ref[...]) / jnp.sqrt(jnp.maximum(deg, 1.0))
    out_ref[...] = _kl_rows(a, wn_ref[...], l_ref[...]) + h0_ref[...]


def _tc_fie1(sp, degp, h0, mu, wn, l):
    return pl.pallas_call(
        _tc_fie1_body,
        grid=(NB,),
        in_specs=[
            pl.BlockSpec((NC, BLK, H), lambda i: (0, i, 0)),
            pl.BlockSpec((NC, BLK, DEGW), lambda i: (0, i, 0)),
            pl.BlockSpec((BLK, H), lambda i: (i, 0)),
            pl.BlockSpec((1, H), lambda i: (0, 0)),
            pl.BlockSpec((H, H), lambda i: (0, 0)),
            pl.BlockSpec((H, H), lambda i: (0, 0)),
        ],
        out_specs=pl.BlockSpec((BLK, H), lambda i: (i, 0)),
        out_shape=jax.ShapeDtypeStruct((N, H), jnp.float32),
    )(sp, degp, h0, mu, wn, l)


def _tc_final_body(sp_ref, degp_ref, h1_ref, b_ref, mu2_ref, mup_ref,
                   wn_ref, l_ref, out_ref, acc_ref):
    i = pl.program_id(0)
    s = sp_ref[0] + sp_ref[1]
    deg = degp_ref[0, :, 0:1] + degp_ref[1, :, 0:1]
    h2 = ((s - deg * mu2_ref[...]) / jnp.sqrt(jnp.maximum(deg, 1.0))
          + h1_ref[...])
    b = b_ref[0, 0, :]
    onehot = (b[None, :] == lax.broadcasted_iota(jnp.int32, (G, BLK), 0)
              ).astype(jnp.bfloat16)
    hcat = jnp.concatenate([h2, jnp.ones_like(h2)], axis=1)
    hi = hcat.astype(jnp.bfloat16)
    lo = (hcat - hi.astype(jnp.float32)).astype(jnp.bfloat16)
    part = (lax.dot_general(onehot, hi, (((1,), (0,)), ((), ())),
                            preferred_element_type=jnp.float32)
            + lax.dot_general(onehot, lo, (((1,), (0,)), ((), ())),
                              preferred_element_type=jnp.float32))

    @pl.when(i == 0)
    def _():
        acc_ref[...] = part

    @pl.when(i > 0)
    def _():
        acc_ref[...] += part

    @pl.when(i == NB - 1)
    def _():
        p = acc_ref[:, :H]
        cnt = acc_ref[:, H:H + 1]
        ap = (p - cnt * mup_ref[...]) * lax.rsqrt(jnp.maximum(cnt, 1.0))
        out_ref[...] = (_kl_rows(ap, wn_ref[...], l_ref[...])
                        + p / jnp.maximum(cnt, 1.0))


def _tc_final(sp, degp, h1, batch3, mu2, mup, wn, l):
    return pl.pallas_call(
        _tc_final_body,
        grid=(NB,),
        in_specs=[
            pl.BlockSpec((NC, BLK, H), lambda i: (0, i, 0)),
            pl.BlockSpec((NC, BLK, DEGW), lambda i: (0, i, 0)),
            pl.BlockSpec((BLK, H), lambda i: (i, 0)),
            pl.BlockSpec((1, 1, BLK), lambda i: (i, 0, 0)),
            pl.BlockSpec((1, H), lambda i: (0, 0)),
            pl.BlockSpec((1, H), lambda i: (0, 0)),
            pl.BlockSpec((H, H), lambda i: (0, 0)),
            pl.BlockSpec((H, H), lambda i: (0, 0)),
        ],
        out_specs=pl.BlockSpec((G, H), lambda i: (0, 0)),
        out_shape=jax.ShapeDtypeStruct((G, H), jnp.float32),
        scratch_shapes=[pltpu.VMEM((G, 2 * H), jnp.float32)],
    )(sp, degp, h1, batch3, mu2, mup, wn, l)


def _prep(w):
    wn = w / jnp.maximum(jnp.linalg.norm(w, axis=1, keepdims=True), 1e-6)
    kmat = jnp.exp(wn @ wn.T - 1.0)
    ev, v = jnp.linalg.eigh(kmat)
    lintrans = (v * (1.0 / jnp.sqrt(jnp.maximum(ev, 1e-6)))) @ v.T
    return wn, lintrans


def kernel(x, edge_index, batch, W0, mu1, Wp1, mu2, mu_pool, Wpool):
    src = edge_index[0].astype(jnp.int32).reshape(NW * NJ, CH)
    dst = edge_index[1].astype(jnp.int32).reshape(NW * NJ, CH)
    batch3 = batch.astype(jnp.int32).reshape(NB, 1, BLK)

    def norm_w(w):
        return w / jnp.maximum(jnp.linalg.norm(w, axis=1, keepdims=True),
                               1e-6)
    wn0, wn1, wnp = norm_w(W0), norm_w(Wp1), norm_w(Wpool)
    kms = jnp.stack([jnp.exp(wn0 @ wn0.T - 1.0),
                     jnp.exp(wn1 @ wn1.T - 1.0),
                     jnp.exp(wnp @ wnp.T - 1.0)])
    ev, v = jnp.linalg.eigh(kms)
    inv = 1.0 / jnp.sqrt(jnp.maximum(ev, 1e-6))
    ls = (v * inv[:, None, :]) @ jnp.swapaxes(v, -1, -2)
    l0, l1, lp = ls[0], ls[1], ls[2]

    h0 = _tc_kl0(x, wn0, l0)
    s1p, degp = _sc_segment_sum(h0, src, dst, want_deg=True)
    h1 = _tc_fie1(s1p, degp, h0, mu1, wn1, l1)
    (s2p,) = _sc_segment_sum(h1, src, dst, want_deg=False)
    return _tc_final(s2p, degp, h1, batch3, mu2, mu_pool, wnp, lp)


# speedup vs baseline: 1.2412x; 1.2412x over previous
import functools

import jax
import jax.numpy as jnp
from jax import lax
from jax.experimental import pallas as pl
from jax.experimental.pallas import tpu as pltpu
from jax.experimental.pallas import tpu_sc as plsc

N = 10000
E = 320000
D = 128
H = 64
G = 64

NC = 2
NS = 16
NW = NC * NS
CH = 125
EPW = E // NW
NJ = EPW // CH
NBUF = 5
RPT = 624
TAILB = RPT * NS
TAILN = N - TAILB
DEGW = 16
ZR = 208

NB = 5
BLK = N // NB


def _sc_body(want_deg, h_hbm, src_hbm, dst_hbm, *rest):
    if want_deg:
        (s_out, deg_out, src_v, dst_v, rows_v, ones_v, zbuf, dzbuf,
         acc_sh, deg_sh, sem0, sem1, sem2, sem3, sem4) = rest
    else:
        (s_out, src_v, dst_v, rows_v, zbuf, acc_sh,
         sem0, sem1, sem2, sem3, sem4) = rest
    sems = (sem0, sem1, sem2, sem3, sem4)
    c = lax.axis_index("c")
    s = lax.axis_index("s")
    w = s * NC + c

    pltpu.sync_copy(src_hbm.at[pl.ds(w * NJ, NJ)], src_v)
    pltpu.sync_copy(dst_hbm.at[pl.ds(w * NJ, NJ)], dst_v)

    def zrow(i, _):
        for cc in range(H // 16):
            zbuf[i, pl.ds(cc * 16, 16)] = jnp.zeros((16,), jnp.float32)
        return 0
    lax.fori_loop(0, ZR, zrow, 0)
    for k in range(RPT // ZR):
        pltpu.sync_copy(zbuf, acc_sh.at[pl.ds(s * RPT + k * ZR, ZR)])

    @pl.when(s == NS - 1)
    def _():
        pltpu.sync_copy(zbuf.at[pl.ds(0, TAILN)], acc_sh.at[pl.ds(TAILB, TAILN)])
    if want_deg:
        def zdrow(i, _):
            dzbuf[i, pl.ds(0, DEGW)] = jnp.zeros((DEGW,), jnp.float32)
            return 0
        lax.fori_loop(0, ZR, zdrow, 0)
        for k in range(RPT // ZR):
            pltpu.sync_copy(dzbuf, deg_sh.at[pl.ds(s * RPT + k * ZR, ZR)])

        @pl.when(s == NS - 1)
        def _():
            pltpu.sync_copy(dzbuf.at[pl.ds(0, TAILN)],
                            deg_sh.at[pl.ds(TAILB, TAILN)])

        def orow(i, _):
            ones_v[i, pl.ds(0, DEGW)] = jnp.ones((DEGW,), jnp.float32)
            return 0
        lax.fori_loop(0, CH, orow, 0)
    plsc.subcore_barrier()

    for b in range(NBUF):
        pltpu.async_copy(h_hbm.at[src_v.at[b]], rows_v.at[b], sems[b])

    def edge_iter(jo, _):
        for b in range(NBUF):
            j = jo * NBUF + b
            pltpu.make_async_copy(h_hbm.at[src_v.at[j]], rows_v.at[b],
                                  sems[b]).wait()
            pltpu.sync_copy(rows_v.at[b], acc_sh.at[dst_v.at[j]], add=True)
            if want_deg:
                pltpu.sync_copy(ones_v, deg_sh.at[dst_v.at[j]], add=True)

            @pl.when(j + NBUF < NJ)
            def _():
                pltpu.async_copy(h_hbm.at[src_v.at[j + NBUF]],
                                 rows_v.at[b], sems[b])
        return 0
    lax.fori_loop(0, NJ // NBUF, edge_iter, 0)
    plsc.subcore_barrier()

    pltpu.sync_copy(acc_sh.at[pl.ds(s * RPT, RPT)],
                    s_out.at[c, pl.ds(s * RPT, RPT)])

    @pl.when(s == NS - 1)
    def _():
        pltpu.sync_copy(acc_sh.at[pl.ds(TAILB, TAILN)],
                        s_out.at[c, pl.ds(TAILB, TAILN)])
    if want_deg:
        pltpu.sync_copy(deg_sh.at[pl.ds(s * RPT, RPT)],
                        deg_out.at[c, pl.ds(s * RPT, RPT)])

        @pl.when(s == NS - 1)
        def _():
            pltpu.sync_copy(deg_sh.at[pl.ds(TAILB, TAILN)],
                            deg_out.at[c, pl.ds(TAILB, TAILN)])


def _sc_segment_sum(h, src2d, dst2d, want_deg):
    mesh = plsc.VectorSubcoreMesh(core_axis_name="c", subcore_axis_name="s")
    out_type = [jax.ShapeDtypeStruct((NC, N, H), jnp.float32)]
    sems = [pltpu.SemaphoreType.DMA] * NBUF
    if want_deg:
        out_type.append(jax.ShapeDtypeStruct((NC, N, DEGW), jnp.float32))
        scratch = [
            pltpu.VMEM((NJ, CH), jnp.int32),
            pltpu.VMEM((NJ, CH), jnp.int32),
            pltpu.VMEM((NBUF, CH, H), jnp.float32),
            pltpu.VMEM((CH, DEGW), jnp.float32),
            pltpu.VMEM((ZR, H), jnp.float32),
            pltpu.VMEM((ZR, DEGW), jnp.float32),
            pltpu.VMEM_SHARED((N, H), jnp.float32),
            pltpu.VMEM_SHARED((N, DEGW), jnp.float32),
        ] + sems
    else:
        scratch = [
            pltpu.VMEM((NJ, CH), jnp.int32),
            pltpu.VMEM((NJ, CH), jnp.int32),
            pltpu.VMEM((NBUF, CH, H), jnp.float32),
            pltpu.VMEM((ZR, H), jnp.float32),
            pltpu.VMEM_SHARED((N, H), jnp.float32),
        ] + sems
    fn = pl.kernel(
        functools.partial(_sc_body, want_deg),
        out_type=tuple(out_type),
        mesh=mesh,
        scratch_types=tuple(scratch),
        compiler_params=pltpu.CompilerParams(use_tc_tiling_on_sc=False),
    )
    return fn(h, src2d, dst2d)


def _kl_rows(a, wn, l):
    nrm = jnp.sqrt(jnp.sum(a * a, axis=1, keepdims=True))
    xh = a / jnp.maximum(nrm, 1e-6)
    t = lax.dot_general(xh, wn, (((1,), (1,)), ((), ())),
                        preferred_element_type=jnp.float32)
    emb = nrm * jnp.exp(t - 1.0)
    return lax.dot_general(emb, l, (((1,), (0,)), ((), ())),
                           preferred_element_type=jnp.float32)


def _tc_kl0_body(x_ref, wn_ref, l_ref, out_ref):
    out_ref[...] = _kl_rows(x_ref[...], wn_ref[...], l_ref[...])


def _tc_kl0(x, wn, l):
    return pl.pallas_call(
        _tc_kl0_body,
        grid=(NB,),
        in_specs=[
            pl.BlockSpec((BLK, D), lambda i: (i, 0)),
            pl.BlockSpec((H, D), lambda i: (0, 0)),
            pl.BlockSpec((H, H), lambda i: (0, 0)),
        ],
        out_specs=pl.BlockSpec((BLK, H), lambda i: (i, 0)),
        out_shape=jax.ShapeDtypeStruct((N, H), jnp.float32),
    )(x, wn, l)


def _tc_fie1_body(sp_ref, degp_ref, h0_ref, mu_ref, wn_ref, l_ref, out_ref):
    s = sp_ref[0] + sp_ref[1]
    deg = degp_ref[0, :, 0:1] + degp_ref[1, :, 0:1]
    a = (s - deg * mu_ref[...]) / jnp.sqrt(jnp.maximum(deg, 1.0))
    out_ref[...] = _kl_rows(a, wn_ref[...], l_ref[...]) + h0_ref[...]


def _tc_fie1(sp, degp, h0, mu, wn, l):
    return pl.pallas_call(
        _tc_fie1_body,
        grid=(NB,),
        in_specs=[
            pl.BlockSpec((NC, BLK, H), lambda i: (0, i, 0)),
            pl.BlockSpec((NC, BLK, DEGW), lambda i: (0, i, 0)),
            pl.BlockSpec((BLK, H), lambda i: (i, 0)),
            pl.BlockSpec((1, H), lambda i: (0, 0)),
            pl.BlockSpec((H, H), lambda i: (0, 0)),
            pl.BlockSpec((H, H), lambda i: (0, 0)),
        ],
        out_specs=pl.BlockSpec((BLK, H), lambda i: (i, 0)),
        out_shape=jax.ShapeDtypeStruct((N, H), jnp.float32),
    )(sp, degp, h0, mu, wn, l)


def _tc_final_body(sp_ref, degp_ref, h1_ref, b_ref, mu2_ref, mup_ref,
                   wn_ref, l_ref, out_ref, acc_ref):
    i = pl.program_id(0)
    s = sp_ref[0] + sp_ref[1]
    deg = degp_ref[0, :, 0:1] + degp_ref[1, :, 0:1]
    h2 = ((s - deg * mu2_ref[...]) / jnp.sqrt(jnp.maximum(deg, 1.0))
          + h1_ref[...])
    b = b_ref[0, 0, :]
    onehot = (b[None, :] == lax.broadcasted_iota(jnp.int32, (G, BLK), 0)
              ).astype(jnp.bfloat16)
    hcat = jnp.concatenate([h2, jnp.ones_like(h2)], axis=1)
    hi = hcat.astype(jnp.bfloat16)
    lo = (hcat - hi.astype(jnp.float32)).astype(jnp.bfloat16)
    part = (lax.dot_general(onehot, hi, (((1,), (0,)), ((), ())),
                            preferred_element_type=jnp.float32)
            + lax.dot_general(onehot, lo, (((1,), (0,)), ((), ())),
                              preferred_element_type=jnp.float32))

    @pl.when(i == 0)
    def _():
        acc_ref[...] = part

    @pl.when(i > 0)
    def _():
        acc_ref[...] += part

    @pl.when(i == NB - 1)
    def _():
        p = acc_ref[:, :H]
        cnt = acc_ref[:, H:H + 1]
        ap = (p - cnt * mup_ref[...]) * lax.rsqrt(jnp.maximum(cnt, 1.0))
        out_ref[...] = (_kl_rows(ap, wn_ref[...], l_ref[...])
                        + p / jnp.maximum(cnt, 1.0))


def _tc_final(sp, degp, h1, batch3, mu2, mup, wn, l):
    return pl.pallas_call(
        _tc_final_body,
        grid=(NB,),
        in_specs=[
            pl.BlockSpec((NC, BLK, H), lambda i: (0, i, 0)),
            pl.BlockSpec((NC, BLK, DEGW), lambda i: (0, i, 0)),
            pl.BlockSpec((BLK, H), lambda i: (i, 0)),
            pl.BlockSpec((1, 1, BLK), lambda i: (i, 0, 0)),
            pl.BlockSpec((1, H), lambda i: (0, 0)),
            pl.BlockSpec((1, H), lambda i: (0, 0)),
            pl.BlockSpec((H, H), lambda i: (0, 0)),
            pl.BlockSpec((H, H), lambda i: (0, 0)),
        ],
        out_specs=pl.BlockSpec((G, H), lambda i: (0, 0)),
        out_shape=jax.ShapeDtypeStruct((G, H), jnp.float32),
        scratch_shapes=[pltpu.VMEM((G, 2 * H), jnp.float32)],
    )(sp, degp, h1, batch3, mu2, mup, wn, l)


def _prep(w):
    wn = w / jnp.maximum(jnp.linalg.norm(w, axis=1, keepdims=True), 1e-6)
    kmat = jnp.exp(wn @ wn.T - 1.0)
    ev, v = jnp.linalg.eigh(kmat)
    lintrans = (v * (1.0 / jnp.sqrt(jnp.maximum(ev, 1e-6)))) @ v.T
    return wn, lintrans


def kernel(x, edge_index, batch, W0, mu1, Wp1, mu2, mu_pool, Wpool):
    src = edge_index[0].astype(jnp.int32).reshape(NW * NJ, CH)
    dst = edge_index[1].astype(jnp.int32).reshape(NW * NJ, CH)
    batch3 = batch.astype(jnp.int32).reshape(NB, 1, BLK)

    wn0, l0 = _prep(W0)
    wn1, l1 = _prep(Wp1)
    wnp, lp = _prep(Wpool)

    h0 = _tc_kl0(x, wn0, l0)
    s1p, degp = _sc_segment_sum(h0, src, dst, want_deg=True)
    h1 = _tc_fie1(s1p, degp, h0, mu1, wn1, l1)
    (s2p,) = _sc_segment_sum(h1, src, dst, want_deg=False)
    return _tc_final(s2p, degp, h1, batch3, mu2, mu_pool, wnp, lp)
